# Initial kernel scaffold; baseline (speedup 1.0000x reference)
import jax
import jax.numpy as jnp
from jax.experimental import pallas as pl


def kernel(atomic_numbers, positions, senders, receivers, embed_table, W1, b1, W2, b2, W3, b3, Wg_s, Wg_v, Wo_s, Wo_v, W_read_s, W_read_v):
    raise NotImplementedError("write your pallas kernel here")


# speedup vs baseline: 6.5293x; 6.5293x over previous
import functools
import jax
import jax.numpy as jnp
from jax import lax
from jax.experimental import pallas as pl
from jax.experimental.pallas import tpu as pltpu
from jax.experimental.pallas import tpu_sc as plsc

N = 10000
E = 160000
EMBED = 128
HID = 64
VEC_OUT = 64

NC = 2
NS = 16
EP = 163840
NP = 10240
B3 = 64
B1 = 512

_HIGH = jax.lax.Precision.HIGHEST


def _p1_body(pos_hbm, snd_hbm, rcv_hbm, relx_hbm, rely_hbm, relz_hbm,
             pos_v, snd_v, rcv_v, ox_v, oy_v, oz_v):
    wid = lax.axis_index("s") * NC + lax.axis_index("c")
    pltpu.sync_copy(pos_hbm, pos_v)
    per_tile = EP // (NC * NS)

    def batch(b, _):
        e0 = wid * per_tile + b * B1
        pltpu.sync_copy(snd_hbm.at[pl.ds(e0, B1)], snd_v)
        pltpu.sync_copy(rcv_hbm.at[pl.ds(e0, B1)], rcv_v)
        for g in range(B1 // 16):
            sl = pl.ds(g * 16, 16)
            s3 = snd_v[sl] * 3
            r3 = rcv_v[sl] * 3
            for d, ref in ((0, ox_v), (1, oy_v), (2, oz_v)):
                ps = plsc.load_gather(pos_v, [s3 + d])
                pr = plsc.load_gather(pos_v, [r3 + d])
                ref[sl] = pr - ps
        pltpu.sync_copy(ox_v, relx_hbm.at[pl.ds(e0, B1)])
        pltpu.sync_copy(oy_v, rely_hbm.at[pl.ds(e0, B1)])
        pltpu.sync_copy(oz_v, relz_hbm.at[pl.ds(e0, B1)])
        return ()

    lax.fori_loop(0, per_tile // B1, batch, ())


def _p1_call(pos_flat, snd, rcv):
    return pl.kernel(
        _p1_body,
        out_type=(jax.ShapeDtypeStruct((EP,), jnp.float32),) * 3,
        mesh=plsc.VectorSubcoreMesh(core_axis_name="c", subcore_axis_name="s"),
        scratch_types=[
            pltpu.VMEM((N * 3,), jnp.float32),
            pltpu.VMEM((B1,), jnp.int32),
            pltpu.VMEM((B1,), jnp.int32),
            pltpu.VMEM((B1,), jnp.float32),
            pltpu.VMEM((B1,), jnp.float32),
            pltpu.VMEM((B1,), jnp.float32),
        ],
        compiler_params=pltpu.CompilerParams(needs_layout_passes=False),
    )(pos_flat, snd, rcv)


def _p2a_body(zf_ref, tab_ref, s4_ref):
    zf = zf_ref[...]
    io = lax.broadcasted_iota(jnp.int32, (1, EMBED), 1).astype(jnp.float32)
    oh = (zf == io).astype(jnp.float32)
    s = jnp.dot(oh, tab_ref[...], precision=_HIGH)
    s4_ref[...] = jnp.stack([s[:, 32 * c:32 * c + 32] for c in range(4)], 0)


def _p2a_call(zf, tab_pad):
    return pl.pallas_call(
        _p2a_body,
        grid=(NP // 1024,),
        in_specs=[
            pl.BlockSpec((1024, 1), lambda i: (i, 0)),
            pl.BlockSpec((EMBED, EMBED), lambda i: (0, 0)),
        ],
        out_specs=pl.BlockSpec((4, 1024, 32), lambda i: (0, i, 0)),
        out_shape=jax.ShapeDtypeStruct((4, NP, 32), jnp.float32),
    )(zf, tab_pad)


def _p2b_body(rx_ref, ry_ref, rz_ref, W1_ref, b1_ref, W2_ref, b2_ref,
              W3_ref, b3_ref, ssP_ref, svP_ref, ux_ref, uy_ref, uz_ref):
    rx, ry, rz = rx_ref[...], ry_ref[...], rz_ref[...]
    norm = jnp.sqrt(rx * rx + ry * ry + rz * rz)
    inv = 1.0 / jnp.maximum(norm, 1e-12)
    ux_ref[...] = rx * inv
    uy_ref[...] = ry * inv
    uz_ref[...] = rz * inv
    h = jax.nn.relu(norm * W1_ref[...] + b1_ref[...])
    h = jax.nn.relu(jnp.dot(h, W2_ref[...], precision=_HIGH) + b2_ref[...])
    scal = jnp.dot(h, W3_ref[...], precision=_HIGH) + b3_ref[...]
    ss, sv = scal[:, :EMBED], scal[:, EMBED:]
    ssP_ref[...] = jnp.stack([ss[:, 32 * c:32 * c + 32] for c in range(4)], 0)
    svP_ref[...] = jnp.stack([sv[:, 32 * c:32 * c + 32] for c in range(4)], 0)


def _p2b_call(rx2, ry2, rz2, W1, b1r, W2, b2r, W3, b3r):
    eb = pl.BlockSpec((1024, 1), lambda i: (i, 0))
    wf = lambda shape: pl.BlockSpec(shape, lambda i: tuple(0 for _ in shape))
    return pl.pallas_call(
        _p2b_body,
        grid=(EP // 1024,),
        in_specs=[eb, eb, eb,
                  wf((1, HID)), wf((1, HID)), wf((HID, HID)), wf((1, HID)),
                  wf((HID, 2 * EMBED)), wf((1, 2 * EMBED))],
        out_specs=[pl.BlockSpec((4, 1024, 32), lambda i: (0, i, 0))] * 2
                  + [eb, eb, eb],
        out_shape=[jax.ShapeDtypeStruct((4, EP, 32), jnp.float32)] * 2
                  + [jax.ShapeDtypeStruct((EP, 1), jnp.float32)] * 3,
    )(rx2, ry2, rz2, W1, b1r, W2, b2r, W3, b3r)


def _p3_body(scomb_hbm, ssP_hbm, svP_hbm, ux_hbm, uy_hbm, uz_hbm,
             snd_hbm, rcv_hbm, zeros_hbm, zeros16_hbm, ones_hbm,
             acc_hbm, cnt_hbm,
             acc_sh, cnt_sh,
             feat_v, ss_v, sv_v, ux_v, uy_v, uz_v,
             snd_v, idx_v, rcv_v, rows_v, ones_v, sem):
    core = lax.axis_index("c")
    sub = lax.axis_index("s")
    rows_per_sub = NP // NS
    per_sub = EP // NS
    nbatch = per_sub // B3

    pltpu.sync_copy(ones_hbm, ones_v)

    def zero_acc():
        for j in range(rows_per_sub // 64):
            pltpu.sync_copy(zeros_hbm, acc_sh.at[pl.ds(sub * rows_per_sub + j * 64, 64)])

    def zero_cnt():
        for j in range(rows_per_sub // 64):
            pltpu.sync_copy(zeros16_hbm, cnt_sh.at[pl.ds(sub * rows_per_sub + j * 64, 64)])

    for k in range(2):
        chunk = core * 2 + k
        zero_acc()
        if k == 0:
            @pl.when(core == 0)
            def _():
                zero_cnt()
        plsc.subcore_barrier()

        def batch(b, _):
            e0 = sub * per_sub + b * B3
            pltpu.sync_copy(snd_hbm.at[pl.ds(e0, B3)], snd_v)
            pltpu.sync_copy(rcv_hbm.at[pl.ds(e0, B3)], rcv_v)
            base = chunk * NP
            for g in range(B3 // 16):
                sl = pl.ds(g * 16, 16)
                idx_v[sl] = snd_v[sl] + base
            pltpu.async_copy(scomb_hbm.at[idx_v], feat_v, sem).wait()
            pltpu.sync_copy(ssP_hbm.at[chunk, pl.ds(e0, B3)], ss_v)
            pltpu.sync_copy(svP_hbm.at[chunk, pl.ds(e0, B3)], sv_v)
            pltpu.sync_copy(ux_hbm.at[pl.ds(e0, B3)], ux_v)
            pltpu.sync_copy(uy_hbm.at[pl.ds(e0, B3)], uy_v)
            pltpu.sync_copy(uz_hbm.at[pl.ds(e0, B3)], uz_v)
            for e in range(B3):
                f0 = feat_v[e, pl.ds(0, 16)]
                f1 = feat_v[e, pl.ds(16, 16)]
                wfs0 = f0 * ss_v[e, pl.ds(0, 16)]
                wfs1 = f1 * ss_v[e, pl.ds(16, 16)]
                wfv0 = f0 * sv_v[e, pl.ds(0, 16)]
                wfv1 = f1 * sv_v[e, pl.ds(16, 16)]
                eidx = jnp.full((16,), e, jnp.int32)
                bux = plsc.load_gather(ux_v, [eidx])
                buy = plsc.load_gather(uy_v, [eidx])
                buz = plsc.load_gather(uz_v, [eidx])
                rows_v[e, pl.ds(0, 16)] = wfs0
                rows_v[e, pl.ds(16, 16)] = wfs1
                rows_v[e, pl.ds(32, 16)] = wfv0 * bux
                rows_v[e, pl.ds(48, 16)] = wfv1 * bux
                rows_v[e, pl.ds(64, 16)] = wfv0 * buy
                rows_v[e, pl.ds(80, 16)] = wfv1 * buy
                rows_v[e, pl.ds(96, 16)] = wfv0 * buz
                rows_v[e, pl.ds(112, 16)] = wfv1 * buz
            pltpu.sync_copy(rows_v, acc_sh.at[rcv_v], add=True)
            if k == 0:
                @pl.when(core == 0)
                def _():
                    pltpu.sync_copy(ones_v, cnt_sh.at[rcv_v], add=True)
            return ()

        lax.fori_loop(0, nbatch, batch, ())
        plsc.subcore_barrier()
        r0 = sub * rows_per_sub
        pltpu.sync_copy(acc_sh.at[pl.ds(r0, rows_per_sub)],
                        acc_hbm.at[chunk, pl.ds(r0, rows_per_sub)])
        if k == 0:
            @pl.when(core == 0)
            def _():
                pltpu.sync_copy(cnt_sh.at[pl.ds(r0, rows_per_sub)],
                                cnt_hbm.at[pl.ds(r0, rows_per_sub)])
        plsc.subcore_barrier()


def _p3_call(s_comb, ssP, svP, ux, uy, uz, snd, rcv):
    zeros = jnp.zeros((64, 128), jnp.float32)
    zeros16 = jnp.zeros((64, 16), jnp.float32)
    ones = jnp.ones((B3, 16), jnp.float32)
    return pl.kernel(
        _p3_body,
        out_type=(jax.ShapeDtypeStruct((4, NP, 128), jnp.float32),
                  jax.ShapeDtypeStruct((NP, 16), jnp.float32)),
        mesh=plsc.VectorSubcoreMesh(core_axis_name="c", subcore_axis_name="s"),
        scratch_types=[
            pltpu.VMEM_SHARED((NP, 128), jnp.float32),
            pltpu.VMEM_SHARED((NP, 16), jnp.float32),
            pltpu.VMEM((B3, 32), jnp.float32),
            pltpu.VMEM((B3, 32), jnp.float32),
            pltpu.VMEM((B3, 32), jnp.float32),
            pltpu.VMEM((B3,), jnp.float32),
            pltpu.VMEM((B3,), jnp.float32),
            pltpu.VMEM((B3,), jnp.float32),
            pltpu.VMEM((B3,), jnp.int32),
            pltpu.VMEM((B3,), jnp.int32),
            pltpu.VMEM((B3,), jnp.int32),
            pltpu.VMEM((B3, 128), jnp.float32),
            pltpu.VMEM((B3, 16), jnp.float32),
            pltpu.SemaphoreType.DMA,
        ],
        compiler_params=pltpu.CompilerParams(needs_layout_passes=False,
                                             use_tc_tiling_on_sc=False),
    )(s_comb, ssP, svP, ux, uy, uz, snd, rcv, zeros, zeros16, ones)


def _p4_body(acc_ref, cnt_ref, s4_ref, Wgs_ref, Wgv_ref, Wos_ref, Wov_ref,
             Wrs_ref, Wrv_ref, out_ref):
    i = pl.program_id(0)
    cnt = jnp.maximum(cnt_ref[:, 0:1], 1.0)
    inv = 1.0 / cnt
    acc = acc_ref[...]
    agg_s = jnp.concatenate([acc[c, :, 0:32] for c in range(4)], 1) * inv
    agg_vx = jnp.concatenate([acc[c, :, 32:64] for c in range(4)], 1) * inv
    agg_vy = jnp.concatenate([acc[c, :, 64:96] for c in range(4)], 1) * inv
    agg_vz = jnp.concatenate([acc[c, :, 96:128] for c in range(4)], 1) * inv
    exp_s = jnp.dot(agg_s, Wgs_ref[...], precision=_HIGH)
    act_s = jax.nn.gelu(exp_s[:, :2 * EMBED])
    gates = jax.nn.sigmoid(exp_s[:, 2 * EMBED:])
    Wgv = Wgv_ref[...]
    gvx = jnp.dot(agg_vx, Wgv, precision=_HIGH) * gates
    gvy = jnp.dot(agg_vy, Wgv, precision=_HIGH) * gates
    gvz = jnp.dot(agg_vz, Wgv, precision=_HIGH) * gates
    skip = jnp.concatenate([s4_ref[c] for c in range(4)], 1)
    cat = jnp.concatenate([act_s, skip], 1)
    out_s = jnp.dot(cat, Wos_ref[...], precision=_HIGH)
    Wov = Wov_ref[...]
    ovx = jnp.dot(gvx, Wov, precision=_HIGH)
    ovy = jnp.dot(gvy, Wov, precision=_HIGH)
    ovz = jnp.dot(gvz, Wov, precision=_HIGH)
    inv_v = ovx * ovx + ovy * ovy + ovz * ovz
    node = (jnp.dot(out_s, Wrs_ref[...], precision=_HIGH)
            + jnp.dot(inv_v, Wrv_ref[...], precision=_HIGH))
    rowid = i * 1024 + lax.broadcasted_iota(jnp.int32, (1024, 1), 0)
    node = jnp.where(rowid < N, node, 0.0)
    psum = jnp.sum(node, keepdims=True).reshape(1, 1)

    @pl.when(i == 0)
    def _():
        out_ref[...] = jnp.zeros((1, 1), jnp.float32)
    out_ref[...] += psum


def _p4_call(acc, cnt, s4, Wg_s, Wg_v, Wo_s, Wo_v, W_read_s, W_read_v):
    wf = lambda shape: pl.BlockSpec(shape, lambda i: tuple(0 for _ in shape))
    return pl.pallas_call(
        _p4_body,
        grid=(NP // 1024,),
        in_specs=[
            pl.BlockSpec((4, 1024, 128), lambda i: (0, i, 0)),
            pl.BlockSpec((1024, 16), lambda i: (i, 0)),
            pl.BlockSpec((4, 1024, 32), lambda i: (0, i, 0)),
            wf((EMBED, 3 * EMBED)), wf((EMBED, EMBED)),
            wf((3 * EMBED, EMBED)), wf((EMBED, VEC_OUT)),
            wf((EMBED, 1)), wf((VEC_OUT, 1)),
        ],
        out_specs=pl.BlockSpec((1, 1), lambda i: (0, 0)),
        out_shape=jax.ShapeDtypeStruct((1, 1), jnp.float32),
    )(acc, cnt, s4, Wg_s, Wg_v, Wo_s, Wo_v, W_read_s, W_read_v)


@jax.jit
def kernel(atomic_numbers, positions, senders, receivers, embed_table,
           W1, b1, W2, b2, W3, b3, Wg_s, Wg_v, Wo_s, Wo_v,
           W_read_s, W_read_v):
    snd_p = jnp.concatenate([senders, jnp.zeros((EP - E,), jnp.int32)])
    rcv_p = jnp.concatenate([receivers, jnp.full((EP - E,), N, jnp.int32)])
    pos_flat = positions.reshape(-1)
    zf = jnp.concatenate([atomic_numbers.astype(jnp.float32),
                          jnp.zeros((NP - N,), jnp.float32)]).reshape(NP, 1)
    tab_pad = jnp.concatenate(
        [embed_table, jnp.zeros((EMBED - embed_table.shape[0], EMBED),
                                jnp.float32)], 0)

    s4 = _p2a_call(zf, tab_pad)
    s_comb = s4.reshape(4 * NP, 32)

    relx, rely, relz = _p1_call(pos_flat, snd_p, rcv_p)

    ssP, svP, ux2, uy2, uz2 = _p2b_call(
        relx.reshape(EP, 1), rely.reshape(EP, 1), relz.reshape(EP, 1),
        W1, b1.reshape(1, HID), W2, b2.reshape(1, HID),
        W3, b3.reshape(1, 2 * EMBED))

    acc, cnt = _p3_call(s_comb, ssP, svP,
                        ux2.reshape(EP), uy2.reshape(EP), uz2.reshape(EP),
                        snd_p, rcv_p)

    total = _p4_call(acc, cnt, s4, Wg_s, Wg_v, Wo_s, Wo_v,
                     W_read_s, W_read_v)
    return total[0, 0] / N
